# baseline (speedup 1.0000x reference)
import functools

import jax
import jax.numpy as jnp
from jax import lax
from jax.experimental import pallas as pl
from jax.experimental.pallas import tpu as pltpu
from jax.experimental.pallas import tpu_sc as plsc

N = 50000
C = 64
CONV = 8
HID = 128
NT = 6
G = 64
HALF = C // 2

R = 3136
NP = 16 * R
NBLK = NP // R
NP2 = NP + R

E_REAL = 800000
CHUNK = 128
KSUB = 2
SUPER = KSUB * CHUNK
NTILES = 16
NSC = 2
EPT = 50176
EP = EPT * NTILES
NSUP = EPT // SUPER
DSUP = EP // (NSC * NTILES) // SUPER
ROWS_PT = NP // NTILES
NZCH = ROWS_PT // CHUNK
NZREM = ROWS_PT - NZCH * CHUNK
DEGW = 16

_MESH = plsc.VectorSubcoreMesh(
    core_axis_name="c", subcore_axis_name="s", num_cores=NSC, num_subcores=NTILES)


def _zero_acc(zeros_hbm, acc_sh, row0, zsem):
    ds = []
    for k in range(NZCH):
        ds.append(pltpu.async_copy(
            zeros_hbm, acc_sh.at[pl.ds(row0 + k * CHUNK, CHUNK)], zsem))
    ds.append(pltpu.async_copy(
        zeros_hbm.at[pl.ds(0, NZREM)],
        acc_sh.at[pl.ds(row0 + NZCH * CHUNK, NZREM)], zsem))
    for d in ds:
        d.wait()


def _write_out(acc_sh, out_view, row0, zsem):
    ds = []
    for k in range(NZCH):
        ds.append(pltpu.async_copy(
            acc_sh.at[pl.ds(row0 + k * CHUNK, CHUNK)],
            out_view.at[pl.ds(row0 + k * CHUNK, CHUNK)], zsem))
    ds.append(pltpu.async_copy(
        acc_sh.at[pl.ds(row0 + NZCH * CHUNK, NZREM)],
        out_view.at[pl.ds(row0 + NZCH * CHUNK, NZREM)], zsem))
    for d in ds:
        d.wait()


def _deg_body(dstm_hbm, zeros_hbm, ones_hbm, deg_out,
              acc_sh, ones_v, dst_sv0, dst_sv1, zsem, ssem0, ssem1, isem0, isem1):
    c = lax.axis_index("c")
    s = lax.axis_index("s")
    row0 = s * ROWS_PT
    _zero_acc(zeros_hbm, acc_sh, row0, zsem)
    pltpu.sync_copy(ones_hbm, ones_v)
    plsc.subcore_barrier()
    erow = c * (EP // NSC // CHUNK) + s * (EP // (NSC * NTILES) // CHUNK)

    def start_idx(sup, dst_sv, sem):
        pltpu.async_copy(dstm_hbm.at[pl.ds(erow + sup * KSUB, KSUB)],
                         dst_sv, sem)

    def wait_idx(dst_sv, sem):
        pltpu.make_async_copy(dstm_hbm.at[pl.ds(0, KSUB)], dst_sv, sem).wait()

    def fire(dst_sv, sem):
        for i in range(KSUB):
            pltpu.async_copy(ones_v, acc_sh.at[dst_sv.at[i]], sem, add=True)

    def drain(dst_sv, sem):
        for i in range(KSUB):
            pltpu.make_async_copy(ones_v, acc_sh.at[dst_sv.at[i]], sem).wait()

    start_idx(0, dst_sv0, isem0)
    start_idx(1, dst_sv1, isem1)

    def body(m2, carry):
        s0 = 2 * m2
        wait_idx(dst_sv0, isem0)
        fire(dst_sv0, ssem0)
        wait_idx(dst_sv1, isem1)
        fire(dst_sv1, ssem1)
        drain(dst_sv0, ssem0)

        @pl.when(s0 + 2 < DSUP)
        def _():
            start_idx(s0 + 2, dst_sv0, isem0)

        drain(dst_sv1, ssem1)

        @pl.when(s0 + 3 < DSUP)
        def _():
            start_idx(s0 + 3, dst_sv1, isem1)

        return carry

    lax.fori_loop(0, DSUP // 2, body, 0)
    plsc.subcore_barrier()
    _write_out(acc_sh, deg_out.at[c], row0, zsem)


_SC_PARAMS = pltpu.CompilerParams(use_tc_tiling_on_sc=False)

_deg_call = functools.partial(
    pl.kernel,
    out_type=jax.ShapeDtypeStruct((NSC, NP, DEGW), jnp.float32),
    mesh=_MESH,
    compiler_params=_SC_PARAMS,
    scratch_types=[
        pltpu.VMEM_SHARED((NP, DEGW), jnp.float32),
        pltpu.VMEM((CHUNK, DEGW), jnp.float32),
        pltpu.VMEM((KSUB, CHUNK), jnp.int32),
        pltpu.VMEM((KSUB, CHUNK), jnp.int32),
        pltpu.SemaphoreType.DMA,
        pltpu.SemaphoreType.DMA,
        pltpu.SemaphoreType.DMA,
        pltpu.SemaphoreType.DMA,
        pltpu.SemaphoreType.DMA,
    ],
)(_deg_body)


def _agg_body(zt_hbm, src2_hbm, dstm_hbm, zeros_hbm, acc_out,
              acc_sh, src_sv0, src_sv1, dst_sv0, dst_sv1, rows_v0, rows_v1,
              zsem, gsem0, gsem1, ssem0, ssem1, isem0, isem1):
    c = lax.axis_index("c")
    s = lax.axis_index("s")
    row0 = s * ROWS_PT
    _zero_acc(zeros_hbm, acc_sh, row0, zsem)
    plsc.subcore_barrier()
    ebase = s * EPT
    erow = ebase // CHUNK

    def load_idx(sup, src_sv, dst_sv):
        pltpu.sync_copy(src2_hbm.at[c, pl.ds(ebase + sup * SUPER, SUPER)],
                        src_sv)
        pltpu.sync_copy(dstm_hbm.at[pl.ds(erow + sup * KSUB, KSUB)], dst_sv)

    zt_tab = zt_hbm.at[c]

    def fire_g(src_sv, rows_v, sem):
        for i in range(KSUB):
            pltpu.async_copy(
                zt_tab.at[src_sv.at[pl.ds(i * CHUNK, CHUNK)]],
                rows_v.at[pl.ds(i * CHUNK, CHUNK)], sem)

    def drain_g(src_sv, rows_v, sem):
        for i in range(KSUB):
            pltpu.make_async_copy(
                zt_tab.at[src_sv.at[pl.ds(i * CHUNK, CHUNK)]],
                rows_v.at[pl.ds(i * CHUNK, CHUNK)], sem).wait()

    def fire_s(rows_v, dst_sv, sem):
        for i in range(KSUB):
            pltpu.async_copy(rows_v.at[pl.ds(i * CHUNK, CHUNK)],
                             acc_sh.at[dst_sv.at[i]], sem, add=True)

    def drain_s(rows_v, dst_sv, sem):
        for i in range(KSUB):
            pltpu.make_async_copy(rows_v.at[pl.ds(i * CHUNK, CHUNK)],
                                  acc_sh.at[dst_sv.at[i]], sem).wait()

    def start_idx(sup, src_sv, dst_sv, sem):
        pltpu.async_copy(src2_hbm.at[c, pl.ds(ebase + sup * SUPER, SUPER)],
                         src_sv, sem)
        pltpu.async_copy(dstm_hbm.at[pl.ds(erow + sup * KSUB, KSUB)],
                         dst_sv, sem)

    def wait_idx(src_sv, dst_sv, sem):
        pltpu.make_async_copy(src2_hbm.at[c, pl.ds(0, SUPER)],
                              src_sv, sem).wait()
        pltpu.make_async_copy(dstm_hbm.at[pl.ds(0, KSUB)],
                              dst_sv, sem).wait()

    load_idx(0, src_sv0, dst_sv0)
    fire_g(src_sv0, rows_v0, gsem0)
    start_idx(1, src_sv1, dst_sv1, isem1)

    def body(m2, carry):
        s0 = 2 * m2
        drain_g(src_sv0, rows_v0, gsem0)
        fire_s(rows_v0, dst_sv0, ssem0)
        wait_idx(src_sv1, dst_sv1, isem1)
        fire_g(src_sv1, rows_v1, gsem1)
        drain_s(rows_v0, dst_sv0, ssem0)

        @pl.when(s0 + 2 < NSUP)
        def _():
            start_idx(s0 + 2, src_sv0, dst_sv0, isem0)

        drain_g(src_sv1, rows_v1, gsem1)
        fire_s(rows_v1, dst_sv1, ssem1)

        @pl.when(s0 + 2 < NSUP)
        def _():
            wait_idx(src_sv0, dst_sv0, isem0)
            fire_g(src_sv0, rows_v0, gsem0)

        drain_s(rows_v1, dst_sv1, ssem1)

        @pl.when(s0 + 3 < NSUP)
        def _():
            start_idx(s0 + 3, src_sv1, dst_sv1, isem1)

        return carry

    lax.fori_loop(0, NSUP // 2, body, 0)
    plsc.subcore_barrier()
    _write_out(acc_sh, acc_out.at[c], row0, zsem)


_agg_call = functools.partial(
    pl.kernel,
    out_type=jax.ShapeDtypeStruct((NSC, NP, HALF), jnp.float32),
    mesh=_MESH,
    compiler_params=_SC_PARAMS,
    scratch_types=[
        pltpu.VMEM_SHARED((NP, HALF), jnp.float32),
        pltpu.VMEM((SUPER,), jnp.int32),
        pltpu.VMEM((SUPER,), jnp.int32),
        pltpu.VMEM((KSUB, CHUNK), jnp.int32),
        pltpu.VMEM((KSUB, CHUNK), jnp.int32),
        pltpu.VMEM((SUPER, HALF), jnp.float32),
        pltpu.VMEM((SUPER, HALF), jnp.float32),
        pltpu.SemaphoreType.DMA,
        pltpu.SemaphoreType.DMA,
        pltpu.SemaphoreType.DMA,
        pltpu.SemaphoreType.DMA,
        pltpu.SemaphoreType.DMA,
        pltpu.SemaphoreType.DMA,
        pltpu.SemaphoreType.DMA,
    ],
)(_agg_body)


def _embed_body(x_ref, deg_ref, emb_ref, h_ref, dinv_ref, st_ref):
    i = pl.program_id(0)
    dg = deg_ref[...]
    deg = (dg[0] + dg[1])[:, 0:1] + 1.0
    dinv = lax.rsqrt(jnp.maximum(deg, 1e-12))
    rid = lax.broadcasted_iota(jnp.int32, (R, 1), 0) + i * R
    dinv = jnp.where(rid < N, dinv, 0.0)
    dinv_ref[...] = dinv
    x = x_ref[...]
    emb = emb_ref[...]
    h = jnp.zeros((R, C), jnp.float32)
    for k in range(NT):
        h = h + jnp.where(x == k, 1.0, 0.0) * emb[k][None, :]
    h = jnp.where(rid < N, h, 0.0)
    h_ref[...] = h
    st_ref[...] = jnp.concatenate([jnp.sum(h, 0), jnp.sum(h * h, 0)])[None, None]


def _embed_call(xp, dega, emb):
    return pl.pallas_call(
        _embed_body,
        grid=(NBLK,),
        in_specs=[pl.BlockSpec((R, 1), lambda i: (i, 0)),
                  pl.BlockSpec((NSC, R, DEGW), lambda i: (0, i, 0)),
                  pl.BlockSpec((NT, C), lambda i: (0, 0))],
        out_specs=[pl.BlockSpec((R, C), lambda i: (i, 0)),
                   pl.BlockSpec((R, 1), lambda i: (i, 0)),
                   pl.BlockSpec((1, 1, 2 * C), lambda i: (i, 0, 0))],
        out_shape=[jax.ShapeDtypeStruct((NP2, C), jnp.float32),
                   jax.ShapeDtypeStruct((NP, 1), jnp.float32),
                   jax.ShapeDtypeStruct((NBLK, 1, 2 * C), jnp.float32)],
    )(xp, dega, emb)


def _fused_body(h_ref, acc_ref, ztp_ref, dinv_ref, b_ref, g_ref, be_ref,
                w_ref, ho_ref, zt_ref, hbuf, stbuf):
    p = pl.program_id(0)
    i = pl.program_id(1)

    @pl.when(p == 0)
    def _():
        a = acc_ref[...]
        zp = ztp_ref[...]
        agg = jnp.concatenate([a[0] + zp[0], a[1] + zp[1]], axis=1)
        hn = jnp.maximum(h_ref[...] + dinv_ref[...] * agg + b_ref[...], 0.0)
        rid = lax.broadcasted_iota(jnp.int32, (R, 1), 0) + i * R
        hn = jnp.where(rid < N, hn, 0.0)
        ho_ref[...] = hn
        hbuf[pl.ds(i * R, R), :] = hn
        stbuf[pl.ds(i, 1), :] = jnp.concatenate(
            [jnp.sum(hn, 0), jnp.sum(hn * hn, 0)])[None]

    @pl.when(p == 1)
    def _():
        st = stbuf[...]
        mean = jnp.sum(st[:, :C], axis=0) / N
        var = jnp.sum(st[:, C:], axis=0) / N - mean * mean
        scale = lax.rsqrt(var + 1e-5) * g_ref[...][0]
        h = hbuf[pl.ds(i * R, R), :]
        zb = (h - mean[None, :]) * scale[None, :] + be_ref[...]
        z = jnp.dot(zb, w_ref[...], preferred_element_type=jnp.float32)
        zt = dinv_ref[...] * z
        zt_ref[...] = jnp.stack([zt[:, :HALF], zt[:, HALF:]], axis=0)


def _fused_call(h, acc, ztp, dinv, b, g, be, w):
    return pl.pallas_call(
        _fused_body,
        grid=(2, NBLK),
        in_specs=[pl.BlockSpec((R, C), lambda p, i: (i * (1 - p), 0)),
                  pl.BlockSpec((NSC, R, HALF),
                               lambda p, i: (0, i * (1 - p), 0)),
                  pl.BlockSpec((NSC, R, HALF),
                               lambda p, i: (0, i * (1 - p), 0)),
                  pl.BlockSpec((R, 1), lambda p, i: (i, 0)),
                  pl.BlockSpec((1, C), lambda p, i: (0, 0)),
                  pl.BlockSpec((1, C), lambda p, i: (0, 0)),
                  pl.BlockSpec((1, C), lambda p, i: (0, 0)),
                  pl.BlockSpec((C, C), lambda p, i: (0, 0))],
        out_specs=[pl.BlockSpec((R, C),
                                lambda p, i: (i * (1 - p) + NBLK * p, 0)),
                   pl.BlockSpec((NSC, R, HALF),
                                lambda p, i: (0, i * p + NBLK * (1 - p), 0))],
        out_shape=[jax.ShapeDtypeStruct((NP2, C), jnp.float32),
                   jax.ShapeDtypeStruct((NSC, NP2, HALF), jnp.float32)],
        scratch_shapes=[pltpu.VMEM((NP, C), jnp.float32),
                        pltpu.VMEM((NBLK, 2 * C), jnp.float32)],
    )(h, acc, ztp, dinv, b, g, be, w)


def _bnmm_body(h_ref, st_ref, g_ref, be_ref, w_ref, dinv_ref, zt_ref):
    st = st_ref[...][:, 0, :]
    mean = jnp.sum(st[:, :C], axis=0) / N
    var = jnp.sum(st[:, C:], axis=0) / N - mean * mean
    scale = lax.rsqrt(var + 1e-5) * g_ref[...][0]
    zb = (h_ref[...] - mean[None, :]) * scale[None, :] + be_ref[...]
    z = jnp.dot(zb, w_ref[...], preferred_element_type=jnp.float32)
    zt = dinv_ref[...] * z
    zt_ref[...] = jnp.stack([zt[:, :HALF], zt[:, HALF:]], axis=0)


def _bnmm_call(h, st, g, be, w, dinv):
    return pl.pallas_call(
        _bnmm_body,
        grid=(NBLK,),
        in_specs=[pl.BlockSpec((R, C), lambda i: (i, 0)),
                  pl.BlockSpec((NBLK, 1, 2 * C), lambda i: (0, 0, 0)),
                  pl.BlockSpec((1, C), lambda i: (0, 0)),
                  pl.BlockSpec((1, C), lambda i: (0, 0)),
                  pl.BlockSpec((C, C), lambda i: (0, 0)),
                  pl.BlockSpec((R, 1), lambda i: (i, 0))],
        out_specs=pl.BlockSpec((NSC, R, HALF), lambda i: (0, i, 0)),
        out_shape=jax.ShapeDtypeStruct((NSC, NP2, HALF), jnp.float32),
    )(h, st, g, be, w, dinv)


def _pool_body(h_ref, acc_ref, ztp_ref, dinv_ref, b_ref, bt_ref,
               ps_ref, cnt_ref):
    a = acc_ref[...]
    zp = ztp_ref[...]
    agg = jnp.concatenate([a[0] + zp[0], a[1] + zp[1]], axis=1)
    hf = jnp.maximum(h_ref[...] + dinv_ref[...] * agg + b_ref[...], 0.0)
    bt = bt_ref[...]
    oh = (bt == lax.broadcasted_iota(jnp.int32, (1, G), 1)).astype(jnp.float32)
    cnt_ref[...] = jnp.sum(oh, axis=0)[None, None]
    ps_ref[...] = lax.dot_general(
        oh, hf, (((0,), (0,)), ((), ())),
        preferred_element_type=jnp.float32)[None]


def _pool_call(h, acc, ztp, dinv, b, batchp):
    return pl.pallas_call(
        _pool_body,
        grid=(NBLK,),
        in_specs=[pl.BlockSpec((R, C), lambda i: (i, 0)),
                  pl.BlockSpec((NSC, R, HALF), lambda i: (0, i, 0)),
                  pl.BlockSpec((NSC, R, HALF), lambda i: (0, i, 0)),
                  pl.BlockSpec((R, 1), lambda i: (i, 0)),
                  pl.BlockSpec((1, C), lambda i: (0, 0)),
                  pl.BlockSpec((R, 1), lambda i: (i, 0))],
        out_specs=[pl.BlockSpec((1, G, C), lambda i: (i, 0, 0)),
                   pl.BlockSpec((1, 1, G), lambda i: (i, 0, 0))],
        out_shape=[jax.ShapeDtypeStruct((NBLK, G, C), jnp.float32),
                   jax.ShapeDtypeStruct((NBLK, 1, G), jnp.float32)],
    )(h, acc, ztp, dinv, b, batchp)


def _head_body(ps_ref, cnt_ref, hw_ref, hb_ref, ow_ref, ob_ref, o_ref):
    pooled = jnp.sum(ps_ref[...], axis=0)
    counts = jnp.sum(cnt_ref[...][:, 0, :], axis=0)
    pooled = pooled / jnp.maximum(counts, 1.0)[:, None]
    hid = jnp.maximum(
        jnp.dot(pooled, hw_ref[...], preferred_element_type=jnp.float32)
        + hb_ref[...], 0.0)
    o_ref[...] = jnp.dot(hid, ow_ref[...],
                         preferred_element_type=jnp.float32) + ob_ref[...]


def _head_call(ps, cnt, hw, hb, ow, ob):
    return pl.pallas_call(
        _head_body,
        out_shape=jax.ShapeDtypeStruct((G, C), jnp.float32),
    )(ps, cnt, hw, hb, ow, ob)


def kernel(x, edge_index, batch, embedding, bn_gamma, bn_beta, conv_W, conv_b,
           hidden_W, hidden_b, out_W, out_b):
    x = x.astype(jnp.int32)
    ei = edge_index.astype(jnp.int32)
    src = jnp.concatenate([ei[0], jnp.zeros((EP - E_REAL,), jnp.int32)])
    dst = jnp.concatenate([ei[1], jnp.full((EP - E_REAL,), N, jnp.int32)])
    src2 = jnp.stack([src, src])
    dstm = dst.reshape(EP // CHUNK, CHUNK)
    zeros_h = jnp.zeros((CHUNK, HALF), jnp.float32)
    zeros_d = jnp.zeros((CHUNK, DEGW), jnp.float32)
    ones_d = jnp.ones((CHUNK, DEGW), jnp.float32)
    xp = jnp.pad(x, (0, NP - N)).reshape(NP, 1)
    batchp = jnp.pad(batch.astype(jnp.int32), (0, NP - N),
                     constant_values=G).reshape(NP, 1)

    dega = _deg_call(dstm, zeros_d, ones_d)
    h, dinv, st = _embed_call(xp, dega, embedding)
    zt = _bnmm_call(h, st, bn_gamma[0].reshape(1, C),
                    bn_beta[0].reshape(1, C), conv_W[0], dinv)
    acc = _agg_call(zt, src2, dstm, zeros_h)
    for i in range(1, CONV):
        h, zt = _fused_call(h, acc, zt, dinv, conv_b[i - 1].reshape(1, C),
                            bn_gamma[i].reshape(1, C),
                            bn_beta[i].reshape(1, C), conv_W[i])
        acc = _agg_call(zt, src2, dstm, zeros_h)
    ps, cnt = _pool_call(h, acc, zt, dinv, conv_b[CONV - 1].reshape(1, C),
                         batchp)
    return _head_call(ps, cnt, hidden_W, hidden_b.reshape(1, HID),
                      out_W, out_b.reshape(1, C))


# speedup vs baseline: 1.0210x; 1.0210x over previous
import functools

import jax
import jax.numpy as jnp
from jax import lax
from jax.experimental import pallas as pl
from jax.experimental.pallas import tpu as pltpu
from jax.experimental.pallas import tpu_sc as plsc

N = 50000
C = 64
CONV = 8
HID = 128
NT = 6
G = 64
HALF = C // 2

R = 3136
NP = 16 * R
NBLK = NP // R
NP2 = NP + R

E_REAL = 800000 + N
CHUNK = 128
KSUB = 2
SUPER = KSUB * CHUNK
NTILES = 16
NSC = 2
EPT = 53248
EP = EPT * NTILES
NSUP = EPT // SUPER
DSUP = EP // (NSC * NTILES) // SUPER
ROWS_PT = NP // NTILES
NZCH = ROWS_PT // CHUNK
NZREM = ROWS_PT - NZCH * CHUNK
DEGW = 16

_MESH = plsc.VectorSubcoreMesh(
    core_axis_name="c", subcore_axis_name="s", num_cores=NSC, num_subcores=NTILES)


def _zero_acc(zeros_hbm, acc_sh, row0, zsem):
    ds = []
    for k in range(NZCH):
        ds.append(pltpu.async_copy(
            zeros_hbm, acc_sh.at[pl.ds(row0 + k * CHUNK, CHUNK)], zsem))
    ds.append(pltpu.async_copy(
        zeros_hbm.at[pl.ds(0, NZREM)],
        acc_sh.at[pl.ds(row0 + NZCH * CHUNK, NZREM)], zsem))
    for d in ds:
        d.wait()


def _write_out(acc_sh, out_view, row0, zsem):
    ds = []
    for k in range(NZCH):
        ds.append(pltpu.async_copy(
            acc_sh.at[pl.ds(row0 + k * CHUNK, CHUNK)],
            out_view.at[pl.ds(row0 + k * CHUNK, CHUNK)], zsem))
    ds.append(pltpu.async_copy(
        acc_sh.at[pl.ds(row0 + NZCH * CHUNK, NZREM)],
        out_view.at[pl.ds(row0 + NZCH * CHUNK, NZREM)], zsem))
    for d in ds:
        d.wait()


def _deg_body(dstm_hbm, zeros_hbm, ones_hbm, deg_out,
              acc_sh, ones_v, dst_sv0, dst_sv1, zsem, ssem0, ssem1, isem0, isem1):
    c = lax.axis_index("c")
    s = lax.axis_index("s")
    row0 = s * ROWS_PT
    _zero_acc(zeros_hbm, acc_sh, row0, zsem)
    pltpu.sync_copy(ones_hbm, ones_v)
    plsc.subcore_barrier()
    erow = c * (EP // NSC // CHUNK) + s * (EP // (NSC * NTILES) // CHUNK)

    def start_idx(sup, dst_sv, sem):
        pltpu.async_copy(dstm_hbm.at[pl.ds(erow + sup * KSUB, KSUB)],
                         dst_sv, sem)

    def wait_idx(dst_sv, sem):
        pltpu.make_async_copy(dstm_hbm.at[pl.ds(0, KSUB)], dst_sv, sem).wait()

    def fire(dst_sv, sem):
        for i in range(KSUB):
            pltpu.async_copy(ones_v, acc_sh.at[dst_sv.at[i]], sem, add=True)

    def drain(dst_sv, sem):
        for i in range(KSUB):
            pltpu.make_async_copy(ones_v, acc_sh.at[dst_sv.at[i]], sem).wait()

    start_idx(0, dst_sv0, isem0)
    start_idx(1, dst_sv1, isem1)

    def body(m2, carry):
        s0 = 2 * m2
        wait_idx(dst_sv0, isem0)
        fire(dst_sv0, ssem0)
        wait_idx(dst_sv1, isem1)
        fire(dst_sv1, ssem1)
        drain(dst_sv0, ssem0)

        @pl.when(s0 + 2 < DSUP)
        def _():
            start_idx(s0 + 2, dst_sv0, isem0)

        drain(dst_sv1, ssem1)

        @pl.when(s0 + 3 < DSUP)
        def _():
            start_idx(s0 + 3, dst_sv1, isem1)

        return carry

    lax.fori_loop(0, DSUP // 2, body, 0)
    plsc.subcore_barrier()
    _write_out(acc_sh, deg_out.at[c], row0, zsem)


_SC_PARAMS = pltpu.CompilerParams(use_tc_tiling_on_sc=False)

_deg_call = functools.partial(
    pl.kernel,
    out_type=jax.ShapeDtypeStruct((NSC, NP, DEGW), jnp.float32),
    mesh=_MESH,
    compiler_params=_SC_PARAMS,
    scratch_types=[
        pltpu.VMEM_SHARED((NP, DEGW), jnp.float32),
        pltpu.VMEM((CHUNK, DEGW), jnp.float32),
        pltpu.VMEM((KSUB, CHUNK), jnp.int32),
        pltpu.VMEM((KSUB, CHUNK), jnp.int32),
        pltpu.SemaphoreType.DMA,
        pltpu.SemaphoreType.DMA,
        pltpu.SemaphoreType.DMA,
        pltpu.SemaphoreType.DMA,
        pltpu.SemaphoreType.DMA,
    ],
)(_deg_body)


def _agg_body(zt_hbm, src2_hbm, dstm_hbm, zeros_hbm, acc_out,
              acc_sh, src_sv0, src_sv1, dst_sv0, dst_sv1, rows_v0, rows_v1,
              zsem, gsem0, gsem1, ssem0, ssem1, isem0, isem1):
    c = lax.axis_index("c")
    s = lax.axis_index("s")
    row0 = s * ROWS_PT
    _zero_acc(zeros_hbm, acc_sh, row0, zsem)
    plsc.subcore_barrier()
    ebase = s * EPT
    erow = ebase // CHUNK

    def load_idx(sup, src_sv, dst_sv):
        pltpu.sync_copy(src2_hbm.at[c, pl.ds(ebase + sup * SUPER, SUPER)],
                        src_sv)
        pltpu.sync_copy(dstm_hbm.at[pl.ds(erow + sup * KSUB, KSUB)], dst_sv)

    zt_tab = zt_hbm.at[c]

    def fire_g(src_sv, rows_v, sem):
        for i in range(KSUB):
            pltpu.async_copy(
                zt_tab.at[src_sv.at[pl.ds(i * CHUNK, CHUNK)]],
                rows_v.at[pl.ds(i * CHUNK, CHUNK)], sem)

    def drain_g(src_sv, rows_v, sem):
        for i in range(KSUB):
            pltpu.make_async_copy(
                zt_tab.at[src_sv.at[pl.ds(i * CHUNK, CHUNK)]],
                rows_v.at[pl.ds(i * CHUNK, CHUNK)], sem).wait()

    def fire_s(rows_v, dst_sv, sem):
        for i in range(KSUB):
            pltpu.async_copy(rows_v.at[pl.ds(i * CHUNK, CHUNK)],
                             acc_sh.at[dst_sv.at[i]], sem, add=True)

    def drain_s(rows_v, dst_sv, sem):
        for i in range(KSUB):
            pltpu.make_async_copy(rows_v.at[pl.ds(i * CHUNK, CHUNK)],
                                  acc_sh.at[dst_sv.at[i]], sem).wait()

    def start_idx(sup, src_sv, dst_sv, sem):
        pltpu.async_copy(src2_hbm.at[c, pl.ds(ebase + sup * SUPER, SUPER)],
                         src_sv, sem)
        pltpu.async_copy(dstm_hbm.at[pl.ds(erow + sup * KSUB, KSUB)],
                         dst_sv, sem)

    def wait_idx(src_sv, dst_sv, sem):
        pltpu.make_async_copy(src2_hbm.at[c, pl.ds(0, SUPER)],
                              src_sv, sem).wait()
        pltpu.make_async_copy(dstm_hbm.at[pl.ds(0, KSUB)],
                              dst_sv, sem).wait()

    load_idx(0, src_sv0, dst_sv0)
    fire_g(src_sv0, rows_v0, gsem0)
    start_idx(1, src_sv1, dst_sv1, isem1)

    def body(m2, carry):
        s0 = 2 * m2
        drain_g(src_sv0, rows_v0, gsem0)
        fire_s(rows_v0, dst_sv0, ssem0)
        wait_idx(src_sv1, dst_sv1, isem1)
        fire_g(src_sv1, rows_v1, gsem1)
        drain_s(rows_v0, dst_sv0, ssem0)

        @pl.when(s0 + 2 < NSUP)
        def _():
            start_idx(s0 + 2, src_sv0, dst_sv0, isem0)

        drain_g(src_sv1, rows_v1, gsem1)
        fire_s(rows_v1, dst_sv1, ssem1)

        @pl.when(s0 + 2 < NSUP)
        def _():
            wait_idx(src_sv0, dst_sv0, isem0)
            fire_g(src_sv0, rows_v0, gsem0)

        drain_s(rows_v1, dst_sv1, ssem1)

        @pl.when(s0 + 3 < NSUP)
        def _():
            start_idx(s0 + 3, src_sv1, dst_sv1, isem1)

        return carry

    lax.fori_loop(0, NSUP // 2, body, 0)
    plsc.subcore_barrier()
    _write_out(acc_sh, acc_out.at[c], row0, zsem)


_agg_call = functools.partial(
    pl.kernel,
    out_type=jax.ShapeDtypeStruct((NSC, NP, HALF), jnp.float32),
    mesh=_MESH,
    compiler_params=_SC_PARAMS,
    scratch_types=[
        pltpu.VMEM_SHARED((NP, HALF), jnp.float32),
        pltpu.VMEM((SUPER,), jnp.int32),
        pltpu.VMEM((SUPER,), jnp.int32),
        pltpu.VMEM((KSUB, CHUNK), jnp.int32),
        pltpu.VMEM((KSUB, CHUNK), jnp.int32),
        pltpu.VMEM((SUPER, HALF), jnp.float32),
        pltpu.VMEM((SUPER, HALF), jnp.float32),
        pltpu.SemaphoreType.DMA,
        pltpu.SemaphoreType.DMA,
        pltpu.SemaphoreType.DMA,
        pltpu.SemaphoreType.DMA,
        pltpu.SemaphoreType.DMA,
        pltpu.SemaphoreType.DMA,
        pltpu.SemaphoreType.DMA,
    ],
)(_agg_body)


def _embed_body(x_ref, deg_ref, emb_ref, h_ref, dinv_ref, st_ref):
    i = pl.program_id(0)
    dg = deg_ref[...]
    deg = (dg[0] + dg[1])[:, 0:1]
    dinv = lax.rsqrt(jnp.maximum(deg, 1e-12))
    rid = lax.broadcasted_iota(jnp.int32, (R, 1), 0) + i * R
    dinv = jnp.where(rid < N, dinv, 0.0)
    dinv_ref[...] = dinv
    x = x_ref[...]
    emb = emb_ref[...]
    h = jnp.zeros((R, C), jnp.float32)
    for k in range(NT):
        h = h + jnp.where(x == k, 1.0, 0.0) * emb[k][None, :]
    h = jnp.where(rid < N, h, 0.0)
    h_ref[...] = h
    st_ref[...] = jnp.concatenate([jnp.sum(h, 0), jnp.sum(h * h, 0)])[None, None]


def _embed_call(xp, dega, emb):
    return pl.pallas_call(
        _embed_body,
        grid=(NBLK,),
        in_specs=[pl.BlockSpec((R, 1), lambda i: (i, 0)),
                  pl.BlockSpec((NSC, R, DEGW), lambda i: (0, i, 0)),
                  pl.BlockSpec((NT, C), lambda i: (0, 0))],
        out_specs=[pl.BlockSpec((R, C), lambda i: (i, 0)),
                   pl.BlockSpec((R, 1), lambda i: (i, 0)),
                   pl.BlockSpec((1, 1, 2 * C), lambda i: (i, 0, 0))],
        out_shape=[jax.ShapeDtypeStruct((NP2, C), jnp.float32),
                   jax.ShapeDtypeStruct((NP, 1), jnp.float32),
                   jax.ShapeDtypeStruct((NBLK, 1, 2 * C), jnp.float32)],
    )(xp, dega, emb)


def _fused_body(h_ref, acc_ref, dinv_ref, b_ref, g_ref, be_ref, w_ref,
                ho_ref, zt_ref, hbuf, stbuf):
    p = pl.program_id(0)
    i = pl.program_id(1)

    @pl.when(p == 0)
    def _():
        a = acc_ref[...]
        agg = jnp.concatenate([a[0], a[1]], axis=1)
        hn = jnp.maximum(h_ref[...] + dinv_ref[...] * agg + b_ref[...], 0.0)
        rid = lax.broadcasted_iota(jnp.int32, (R, 1), 0) + i * R
        hn = jnp.where(rid < N, hn, 0.0)
        ho_ref[...] = hn
        hbuf[pl.ds(i * R, R), :] = hn
        stbuf[pl.ds(i, 1), :] = jnp.concatenate(
            [jnp.sum(hn, 0), jnp.sum(hn * hn, 0)])[None]

    @pl.when(p == 1)
    def _():
        st = stbuf[...]
        mean = jnp.sum(st[:, :C], axis=0) / N
        var = jnp.sum(st[:, C:], axis=0) / N - mean * mean
        scale = lax.rsqrt(var + 1e-5) * g_ref[...][0]
        h = hbuf[pl.ds(i * R, R), :]
        zb = (h - mean[None, :]) * scale[None, :] + be_ref[...]
        z = jnp.dot(zb, w_ref[...], preferred_element_type=jnp.float32)
        zt = dinv_ref[...] * z
        zt_ref[...] = jnp.stack([zt[:, :HALF], zt[:, HALF:]], axis=0)


def _fused_call(h, acc, dinv, b, g, be, w):
    return pl.pallas_call(
        _fused_body,
        grid=(2, NBLK),
        in_specs=[pl.BlockSpec((R, C), lambda p, i: (i * (1 - p), 0)),
                  pl.BlockSpec((NSC, R, HALF),
                               lambda p, i: (0, i * (1 - p), 0)),
                  pl.BlockSpec((R, 1), lambda p, i: (i, 0)),
                  pl.BlockSpec((1, C), lambda p, i: (0, 0)),
                  pl.BlockSpec((1, C), lambda p, i: (0, 0)),
                  pl.BlockSpec((1, C), lambda p, i: (0, 0)),
                  pl.BlockSpec((C, C), lambda p, i: (0, 0))],
        out_specs=[pl.BlockSpec((R, C),
                                lambda p, i: (i * (1 - p) + NBLK * p, 0)),
                   pl.BlockSpec((NSC, R, HALF),
                                lambda p, i: (0, i * p + NBLK * (1 - p), 0))],
        out_shape=[jax.ShapeDtypeStruct((NP2, C), jnp.float32),
                   jax.ShapeDtypeStruct((NSC, NP2, HALF), jnp.float32)],
        scratch_shapes=[pltpu.VMEM((NP, C), jnp.float32),
                        pltpu.VMEM((NBLK, 2 * C), jnp.float32)],
    )(h, acc, dinv, b, g, be, w)


def _bnmm_body(h_ref, st_ref, g_ref, be_ref, w_ref, dinv_ref, zt_ref):
    st = st_ref[...][:, 0, :]
    mean = jnp.sum(st[:, :C], axis=0) / N
    var = jnp.sum(st[:, C:], axis=0) / N - mean * mean
    scale = lax.rsqrt(var + 1e-5) * g_ref[...][0]
    zb = (h_ref[...] - mean[None, :]) * scale[None, :] + be_ref[...]
    z = jnp.dot(zb, w_ref[...], preferred_element_type=jnp.float32)
    zt = dinv_ref[...] * z
    zt_ref[...] = jnp.stack([zt[:, :HALF], zt[:, HALF:]], axis=0)


def _bnmm_call(h, st, g, be, w, dinv):
    return pl.pallas_call(
        _bnmm_body,
        grid=(NBLK,),
        in_specs=[pl.BlockSpec((R, C), lambda i: (i, 0)),
                  pl.BlockSpec((NBLK, 1, 2 * C), lambda i: (0, 0, 0)),
                  pl.BlockSpec((1, C), lambda i: (0, 0)),
                  pl.BlockSpec((1, C), lambda i: (0, 0)),
                  pl.BlockSpec((C, C), lambda i: (0, 0)),
                  pl.BlockSpec((R, 1), lambda i: (i, 0))],
        out_specs=pl.BlockSpec((NSC, R, HALF), lambda i: (0, i, 0)),
        out_shape=jax.ShapeDtypeStruct((NSC, NP2, HALF), jnp.float32),
    )(h, st, g, be, w, dinv)


def _pool_body(h_ref, acc_ref, dinv_ref, b_ref, bt_ref, ps_ref, cnt_ref):
    a = acc_ref[...]
    agg = jnp.concatenate([a[0], a[1]], axis=1)
    hf = jnp.maximum(h_ref[...] + dinv_ref[...] * agg + b_ref[...], 0.0)
    bt = bt_ref[...]
    oh = (bt == lax.broadcasted_iota(jnp.int32, (1, G), 1)).astype(jnp.float32)
    cnt_ref[...] = jnp.sum(oh, axis=0)[None, None]
    ps_ref[...] = lax.dot_general(
        oh, hf, (((0,), (0,)), ((), ())),
        preferred_element_type=jnp.float32)[None]


def _pool_call(h, acc, dinv, b, batchp):
    return pl.pallas_call(
        _pool_body,
        grid=(NBLK,),
        in_specs=[pl.BlockSpec((R, C), lambda i: (i, 0)),
                  pl.BlockSpec((NSC, R, HALF), lambda i: (0, i, 0)),
                  pl.BlockSpec((R, 1), lambda i: (i, 0)),
                  pl.BlockSpec((1, C), lambda i: (0, 0)),
                  pl.BlockSpec((R, 1), lambda i: (i, 0))],
        out_specs=[pl.BlockSpec((1, G, C), lambda i: (i, 0, 0)),
                   pl.BlockSpec((1, 1, G), lambda i: (i, 0, 0))],
        out_shape=[jax.ShapeDtypeStruct((NBLK, G, C), jnp.float32),
                   jax.ShapeDtypeStruct((NBLK, 1, G), jnp.float32)],
    )(h, acc, dinv, b, batchp)


def _head_body(ps_ref, cnt_ref, hw_ref, hb_ref, ow_ref, ob_ref, o_ref):
    pooled = jnp.sum(ps_ref[...], axis=0)
    counts = jnp.sum(cnt_ref[...][:, 0, :], axis=0)
    pooled = pooled / jnp.maximum(counts, 1.0)[:, None]
    hid = jnp.maximum(
        jnp.dot(pooled, hw_ref[...], preferred_element_type=jnp.float32)
        + hb_ref[...], 0.0)
    o_ref[...] = jnp.dot(hid, ow_ref[...],
                         preferred_element_type=jnp.float32) + ob_ref[...]


def _head_call(ps, cnt, hw, hb, ow, ob):
    return pl.pallas_call(
        _head_body,
        out_shape=jax.ShapeDtypeStruct((G, C), jnp.float32),
    )(ps, cnt, hw, hb, ow, ob)


def kernel(x, edge_index, batch, embedding, bn_gamma, bn_beta, conv_W, conv_b,
           hidden_W, hidden_b, out_W, out_b):
    x = x.astype(jnp.int32)
    ei = edge_index.astype(jnp.int32)
    loops = jnp.arange(N, dtype=jnp.int32)
    src = jnp.concatenate([ei[0], loops,
                           jnp.zeros((EP - E_REAL,), jnp.int32)])
    dst = jnp.concatenate([ei[1], loops,
                           jnp.full((EP - E_REAL,), N, jnp.int32)])
    src2 = jnp.stack([src, src])
    dstm = dst.reshape(EP // CHUNK, CHUNK)
    zeros_h = jnp.zeros((CHUNK, HALF), jnp.float32)
    zeros_d = jnp.zeros((CHUNK, DEGW), jnp.float32)
    ones_d = jnp.ones((CHUNK, DEGW), jnp.float32)
    xp = jnp.pad(x, (0, NP - N)).reshape(NP, 1)
    batchp = jnp.pad(batch.astype(jnp.int32), (0, NP - N),
                     constant_values=G).reshape(NP, 1)

    dega = _deg_call(dstm, zeros_d, ones_d)
    h, dinv, st = _embed_call(xp, dega, embedding)
    zt = _bnmm_call(h, st, bn_gamma[0].reshape(1, C),
                    bn_beta[0].reshape(1, C), conv_W[0], dinv)
    acc = _agg_call(zt, src2, dstm, zeros_h)
    for i in range(1, CONV):
        h, zt = _fused_call(h, acc, dinv, conv_b[i - 1].reshape(1, C),
                            bn_gamma[i].reshape(1, C),
                            bn_beta[i].reshape(1, C), conv_W[i])
        acc = _agg_call(zt, src2, dstm, zeros_h)
    ps, cnt = _pool_call(h, acc, dinv, conv_b[CONV - 1].reshape(1, C), batchp)
    return _head_call(ps, cnt, hidden_W, hidden_b.reshape(1, HID),
                      out_W, out_b.reshape(1, C))
